# baseline (speedup 1.0000x reference)
import jax
import jax.numpy as jnp
from jax.experimental import pallas as pl
from jax.experimental.pallas import tpu as pltpu

D_MODEL = 1024
NUM_EXPERTS = 8
TOP_K = 2
R = 16
SCALING = 32.0 / 16.0

ER = NUM_EXPERTS * R
BLOCK_N = 2048


def _fused_kernel(x_ref, w_ref, b_ref, wr_ref, a_ref, bb_ref, o_ref):
    xb = x_ref[...]
    base = jax.lax.dot_general(
        xb, w_ref[...], (((1,), (1,)), ((), ())),
        preferred_element_type=jnp.float32)
    base = base + b_ref[...]

    logitsT = jax.lax.dot_general(
        wr_ref[...], xb, (((1,), (1,)), ((), ())),
        preferred_element_type=jnp.float32)
    bn = xb.shape[0]

    m1 = jnp.max(logitsT, axis=0, keepdims=True)
    m2 = jnp.max(jnp.where(logitsT == m1, -jnp.inf, logitsT),
                 axis=0, keepdims=True)
    denom = 1.0 + jnp.exp(m2 - m1)
    gate8T = jnp.where(logitsT >= m2,
                       jnp.exp(logitsT - m1), 0.0) / denom

    erow = jax.lax.broadcasted_iota(jnp.int32, (NUM_EXPERTS, ER), 0)
    ecol = jax.lax.broadcasted_iota(jnp.int32, (NUM_EXPERTS, ER), 1) // R
    expand = (erow == ecol).astype(jnp.float32) * SCALING
    gates = jax.lax.dot_general(
        gate8T, expand, (((0,), (0,)), ((), ())),
        preferred_element_type=jnp.float32)

    hidden = jax.lax.dot_general(
        xb, a_ref[...], (((1,), (0,)), ((), ())),
        preferred_element_type=jnp.float32)
    hidden = hidden * gates
    delta = jax.lax.dot_general(
        hidden, bb_ref[...], (((1,), (0,)), ((), ())),
        preferred_element_type=jnp.float32)
    o_ref[...] = base + delta


def kernel(x, W_base, b_base, W_router, lora_A, lora_B):
    n, d = x.shape
    A_all = lora_A.transpose(1, 0, 2).reshape(d, ER)
    B_all = lora_B.reshape(ER, d)
    b2 = b_base.reshape(1, d)

    grid = (n // BLOCK_N,)
    out = pl.pallas_call(
        _fused_kernel,
        grid=grid,
        in_specs=[
            pl.BlockSpec((BLOCK_N, d), lambda i: (i, 0)),
            pl.BlockSpec((d, d), lambda i: (0, 0)),
            pl.BlockSpec((1, d), lambda i: (0, 0)),
            pl.BlockSpec((NUM_EXPERTS, d), lambda i: (0, 0)),
            pl.BlockSpec((d, ER), lambda i: (0, 0)),
            pl.BlockSpec((ER, d), lambda i: (0, 0)),
        ],
        out_specs=pl.BlockSpec((BLOCK_N, d), lambda i: (i, 0)),
        out_shape=jax.ShapeDtypeStruct((n, d), x.dtype),
        compiler_params=pltpu.CompilerParams(
            dimension_semantics=("parallel",)),
    )(x, W_base, b2, W_router, A_all, B_all)
    return out


# speedup vs baseline: 1.0134x; 1.0134x over previous
import jax
import jax.numpy as jnp
from jax.experimental import pallas as pl
from jax.experimental.pallas import tpu as pltpu

D_MODEL = 1024
NUM_EXPERTS = 8
TOP_K = 2
R = 16
SCALING = 32.0 / 16.0

ER = NUM_EXPERTS * R
BLOCK_N = 1024


def _fused_kernel(x_ref, w_ref, b_ref, wr_ref, a_ref, bb_ref, o_ref):
    xb = x_ref[...]
    base = jax.lax.dot_general(
        xb, w_ref[...], (((1,), (1,)), ((), ())),
        preferred_element_type=jnp.float32)
    base = base + b_ref[...]

    logitsT = jax.lax.dot_general(
        wr_ref[...], xb, (((1,), (1,)), ((), ())),
        preferred_element_type=jnp.float32)
    bn = xb.shape[0]

    m1 = jnp.max(logitsT, axis=0, keepdims=True)
    m2 = jnp.max(jnp.where(logitsT == m1, -jnp.inf, logitsT),
                 axis=0, keepdims=True)
    denom = 1.0 + jnp.exp(m2 - m1)
    gate8T = jnp.where(logitsT >= m2,
                       jnp.exp(logitsT - m1), 0.0) / denom

    erow = jax.lax.broadcasted_iota(jnp.int32, (NUM_EXPERTS, ER), 0)
    ecol = jax.lax.broadcasted_iota(jnp.int32, (NUM_EXPERTS, ER), 1) // R
    expand = (erow == ecol).astype(jnp.float32) * SCALING
    gates = jax.lax.dot_general(
        gate8T, expand, (((0,), (0,)), ((), ())),
        preferred_element_type=jnp.float32)

    hidden = jax.lax.dot_general(
        xb, a_ref[...], (((1,), (0,)), ((), ())),
        preferred_element_type=jnp.float32)
    hidden = hidden * gates
    delta = jax.lax.dot_general(
        hidden, bb_ref[...], (((1,), (0,)), ((), ())),
        preferred_element_type=jnp.float32)
    o_ref[...] = base + delta


def kernel(x, W_base, b_base, W_router, lora_A, lora_B):
    n, d = x.shape
    A_all = lora_A.transpose(1, 0, 2).reshape(d, ER)
    B_all = lora_B.reshape(ER, d)
    b2 = b_base.reshape(1, d)

    grid = (n // BLOCK_N,)
    out = pl.pallas_call(
        _fused_kernel,
        grid=grid,
        in_specs=[
            pl.BlockSpec((BLOCK_N, d), lambda i: (i, 0)),
            pl.BlockSpec((d, d), lambda i: (0, 0)),
            pl.BlockSpec((1, d), lambda i: (0, 0)),
            pl.BlockSpec((NUM_EXPERTS, d), lambda i: (0, 0)),
            pl.BlockSpec((d, ER), lambda i: (0, 0)),
            pl.BlockSpec((ER, d), lambda i: (0, 0)),
        ],
        out_specs=pl.BlockSpec((BLOCK_N, d), lambda i: (i, 0)),
        out_shape=jax.ShapeDtypeStruct((n, d), x.dtype),
        compiler_params=pltpu.CompilerParams(
            dimension_semantics=("arbitrary",)),
    )(x, W_base, b2, W_router, A_all, B_all)
    return out
